# Initial kernel scaffold; baseline (speedup 1.0000x reference)
import jax
import jax.numpy as jnp
from jax.experimental import pallas as pl


def kernel(patches, forward_indexes):
    raise NotImplementedError("write your pallas kernel here")


# speedup vs baseline: 1.2283x; 1.2283x over previous
import functools

import jax
import jax.numpy as jnp
from jax import lax
from jax.experimental import pallas as pl
from jax.experimental.pallas import tpu as pltpu
from jax.experimental.pallas import tpu_sc as plsc

T = 576
B = 64
C = 768
KEEP = 144

NC = 2
NS = 16
NW = NC * NS

N_FWD = T * B
N_VIS = KEEP * B
FWD_PER_W = N_FWD // NW
VIS_PER_W = N_VIS // NW
ROWS_PER_FWD_W = T // NW

CHUNK = 48
N_CHUNK = VIS_PER_W // CHUNK


def _body(patches_hbm, fwd_hbm, vis_hbm, bwd_hbm,
          fwd_a, sidx, sval, fwd_b, gidx, buf0, buf1,
          sem_s, sem_g, sem_w0, sem_w1):
    wid = lax.axis_index("s") * NC + lax.axis_index("c")
    lane = lax.iota(jnp.int32, 16)

    pltpu.sync_copy(fwd_hbm.at[pl.ds(wid * FWD_PER_W, FWD_PER_W)], fwd_a)

    def body_a(j, carry):
        f = fwd_a[pl.ds(j * 16, 16)]
        b = (j % 4) * 16 + lane
        row = wid * ROWS_PER_FWD_W + j // 4
        sidx[pl.ds(j * 16, 16)] = f * B + b
        sval[pl.ds(j * 16, 16)] = jnp.full((16,), 0, jnp.int32) + row
        return carry

    lax.fori_loop(0, FWD_PER_W // 16, body_a, 0)
    scat = pltpu.async_copy(sval, bwd_hbm.at[sidx], sem_s)

    pltpu.sync_copy(fwd_hbm.at[pl.ds(wid * VIS_PER_W, VIS_PER_W)], fwd_b)

    def body_b(j, carry):
        f = fwd_b[pl.ds(j * 16, 16)]
        boff = (wid * VIS_PER_W + j * 16) % B
        gidx[j // 3, pl.ds((j % 3) * 16, 16)] = f * B + boff + lane
        return carry

    lax.fori_loop(0, VIS_PER_W // 16, body_b, 0)

    bufs = (buf0, buf1)
    wsems = (sem_w0, sem_w1)
    pending = [None, None]
    for k in range(N_CHUNK):
        i_buf = k % 2
        if pending[i_buf] is not None:
            pending[i_buf].wait()
        g = pltpu.async_copy(patches_hbm.at[gidx.at[k]], bufs[i_buf], sem_g)
        g.wait()
        w = pltpu.async_copy(
            bufs[i_buf],
            vis_hbm.at[pl.ds(wid * VIS_PER_W + k * CHUNK, CHUNK)],
            wsems[i_buf])
        pending[i_buf] = w
    for w in pending:
        w.wait()
    scat.wait()


@functools.partial(
    pl.kernel,
    out_type=[
        jax.ShapeDtypeStruct((N_VIS, C), jnp.float32),
        jax.ShapeDtypeStruct((N_FWD,), jnp.int32),
    ],
    mesh=plsc.VectorSubcoreMesh(core_axis_name="c", subcore_axis_name="s"),
    scratch_types=[
        pltpu.VMEM((FWD_PER_W,), jnp.int32),
        pltpu.VMEM((FWD_PER_W,), jnp.int32),
        pltpu.VMEM((FWD_PER_W,), jnp.int32),
        pltpu.VMEM((VIS_PER_W,), jnp.int32),
        pltpu.VMEM((N_CHUNK, CHUNK), jnp.int32),
        pltpu.VMEM((CHUNK, C), jnp.float32),
        pltpu.VMEM((CHUNK, C), jnp.float32),
        pltpu.SemaphoreType.DMA,
        pltpu.SemaphoreType.DMA,
        pltpu.SemaphoreType.DMA,
        pltpu.SemaphoreType.DMA,
    ],
)
def _patch_shuffle(patches_hbm, fwd_hbm, vis_hbm, bwd_hbm, *rest):
    _body(patches_hbm, fwd_hbm, vis_hbm, bwd_hbm, *rest)


def kernel(patches, forward_indexes):
    p_flat = patches.reshape(T * B, C)
    f_flat = forward_indexes.reshape(N_FWD)
    vis_flat, bwd_flat = _patch_shuffle(p_flat, f_flat)
    return (vis_flat.reshape(KEEP, B, C), forward_indexes,
            bwd_flat.reshape(T, B))
